# Initial kernel scaffold; baseline (speedup 1.0000x reference)
import jax
import jax.numpy as jnp
from jax.experimental import pallas as pl


def kernel(xA, edge_indexA, edge_attrA, batchA, xB, edge_indexB, edge_attrB, batchB, context, params):
    raise NotImplementedError("write your pallas kernel here")


# speedup vs baseline: 2.7071x; 2.7071x over previous
import functools

import jax
import jax.numpy as jnp
from jax import lax
from jax.experimental import pallas as pl
from jax.experimental.pallas import tpu as pltpu
from jax.experimental.pallas import tpu_sc as plsc

EPS = 1e-5
F32 = jnp.float32


_NSUB = 16
_KCH = 128


def _sc_scatter(h, src, dst):
  N = h.shape[0]
  E = src.shape[0]
  hp = jnp.pad(h, ((0, 0), (0, 84)))
  h0 = hp[:, :128]
  h1 = hp[:, 128:256]
  h2 = hp[:, 256:]
  rpw = (N // _NSUB + 7) // 8 * 8
  npad = rpw * _NSUB
  z = jnp.zeros((rpw, 128), F32)

  epw1 = E // _NSUB
  nfull1, rem1 = divmod(epw1, _KCH)
  epw2 = E // 2 // _NSUB
  nfull2, rem2 = divmod(epw2, _KCH)

  mesh = plsc.VectorSubcoreMesh(core_axis_name="c", subcore_axis_name="s")

  @functools.partial(
      pl.kernel,
      mesh=mesh,
      out_type=(jax.ShapeDtypeStruct((2, npad, 128), F32),
                jax.ShapeDtypeStruct((2, npad, 128), F32)),
      scratch_types=[
          pltpu.VMEM((_KCH,), jnp.int32),
          pltpu.VMEM((_KCH,), jnp.int32),
          pltpu.VMEM((_KCH, 128), F32),
          pltpu.VMEM_SHARED((npad, 128), F32),
          pltpu.SemaphoreType.DMA,
      ],
  )
  def k(h0_hbm, h1_hbm, h2_hbm, src_hbm, dst_hbm, z_hbm, out1_hbm, out2_hbm,
        src_v, dst_v, rows_v, acc_sh, sem):
    c = lax.axis_index("c")
    s = lax.axis_index("s")

    def chunk(tbl, base, n):
      pltpu.sync_copy(src_hbm.at[pl.ds(base, n)], src_v.at[pl.ds(0, n)])
      pltpu.sync_copy(dst_hbm.at[pl.ds(base, n)], dst_v.at[pl.ds(0, n)])
      pltpu.async_copy(tbl.at[src_v.at[pl.ds(0, n)]],
                       rows_v.at[pl.ds(0, n)], sem).wait()
      pltpu.sync_copy(rows_v.at[pl.ds(0, n)],
                      acc_sh.at[dst_v.at[pl.ds(0, n)]], add=True)

    def sweep(tbl, estart, nfull, rem, out_ref):
      pltpu.sync_copy(z_hbm, acc_sh.at[pl.ds(s * rpw, rpw)])
      plsc.subcore_barrier()
      epw = nfull * _KCH + rem

      def body(i, carry):
        chunk(tbl, estart + s * epw + i * _KCH, _KCH)
        return carry

      lax.fori_loop(0, nfull, body, 0)
      if rem:
        chunk(tbl, estart + s * epw + nfull * _KCH, rem)
      plsc.subcore_barrier()
      pltpu.sync_copy(acc_sh.at[pl.ds(s * rpw, rpw)],
                      out_ref.at[c, pl.ds(s * rpw, rpw)])
      plsc.subcore_barrier()

    @pl.when(c == 0)
    def _():
      sweep(h0_hbm, 0, nfull1, rem1, out1_hbm)

    @pl.when(c == 1)
    def _():
      sweep(h1_hbm, 0, nfull1, rem1, out1_hbm)

    sweep(h2_hbm, c * (E // 2), nfull2, rem2, out2_hbm)

  out1, out2 = k(h0, h1, h2, src, dst, z)
  s01 = jnp.concatenate([out1[0, :N], out1[1, :N]], axis=1)
  return s01, out2[0, :N], out2[1, :N]


def _sc_hist(a0, a1, dst, n_nodes):
  N = n_nodes
  E = dst.shape[0]
  T1 = jnp.eye(5, 128, dtype=F32)
  T2 = jnp.eye(3, 128, dtype=F32)
  epw = E // _NSUB
  nfull, rem = divmod(epw, _KCH)
  rpw = (N // _NSUB + 7) // 8 * 8
  npad = rpw * _NSUB
  z = jnp.zeros((rpw, 128), F32)

  mesh = plsc.VectorSubcoreMesh(core_axis_name="c", subcore_axis_name="s")

  @functools.partial(
      pl.kernel,
      mesh=mesh,
      out_type=jax.ShapeDtypeStruct((2, npad, 128), F32),
      scratch_types=[
          pltpu.VMEM((_KCH,), jnp.int32),
          pltpu.VMEM((_KCH,), jnp.int32),
          pltpu.VMEM((_KCH, 128), F32),
          pltpu.VMEM_SHARED((npad, 128), F32),
          pltpu.SemaphoreType.DMA,
      ],
  )
  def k(t1_hbm, t2_hbm, a0_hbm, a1_hbm, dst_hbm, z_hbm, out_hbm,
        idx_v, dst_v, rows_v, acc_sh, sem):
    c = lax.axis_index("c")
    s = lax.axis_index("s")
    pltpu.sync_copy(z_hbm, acc_sh.at[pl.ds(s * rpw, rpw)])
    plsc.subcore_barrier()

    def chunk(base, n):
      @pl.when(c == 0)
      def _():
        pltpu.sync_copy(a0_hbm.at[pl.ds(base, n)], idx_v.at[pl.ds(0, n)])
        pltpu.async_copy(t1_hbm.at[idx_v.at[pl.ds(0, n)]],
                         rows_v.at[pl.ds(0, n)], sem).wait()

      @pl.when(c == 1)
      def _():
        pltpu.sync_copy(a1_hbm.at[pl.ds(base, n)], idx_v.at[pl.ds(0, n)])
        pltpu.async_copy(t2_hbm.at[idx_v.at[pl.ds(0, n)]],
                         rows_v.at[pl.ds(0, n)], sem).wait()

      pltpu.sync_copy(dst_hbm.at[pl.ds(base, n)], dst_v.at[pl.ds(0, n)])
      pltpu.sync_copy(rows_v.at[pl.ds(0, n)],
                      acc_sh.at[dst_v.at[pl.ds(0, n)]], add=True)

    def body(i, carry):
      chunk(s * epw + i * _KCH, _KCH)
      return carry

    lax.fori_loop(0, nfull, body, 0)
    if rem:
      chunk(s * epw + nfull * _KCH, rem)

    plsc.subcore_barrier()
    pltpu.sync_copy(acc_sh.at[pl.ds(s * rpw, rpw)],
                    out_hbm.at[c, pl.ds(s * rpw, rpw)])

  out = k(T1, T2, a0, a1, dst, z)
  return out[0, :N, :16], out[1, :N, :16]


def _rb(t):
  return t.astype(jnp.bfloat16).astype(F32)


def _act(y, act):
  if act == "relu":
    return jnp.maximum(y, 0.0)
  if act == "leaky":
    return jnp.where(y >= 0, y, 0.01 * y)
  return y


def _row_block(n):
  if n % 400 == 0:
    return 400
  return n


def _mm(x, w, b, act=None):
  M, K = x.shape
  Nc = w.shape[1]
  bm = _row_block(M)

  def body(x_ref, w_ref, b_ref, o_ref):
    y = jnp.dot(_rb(x_ref[...]), _rb(w_ref[...]), preferred_element_type=F32)
    o_ref[...] = _act(y + b_ref[...], act)

  return pl.pallas_call(
      body,
      grid=(M // bm,),
      in_specs=[
          pl.BlockSpec((bm, K), lambda i: (i, 0)),
          pl.BlockSpec((K, Nc), lambda i: (0, 0)),
          pl.BlockSpec((1, Nc), lambda i: (0, 0)),
      ],
      out_specs=pl.BlockSpec((bm, Nc), lambda i: (i, 0)),
      out_shape=jax.ShapeDtypeStruct((M, Nc), F32),
  )(x, w, b.reshape(1, Nc))


def _embed(x0, x1, t1, t2):
  NB, _, bm = x0.shape
  V1, D = t1.shape
  V2 = t2.shape[0]

  def body(x0_ref, x1_ref, t1_ref, t2_ref, o_ref):
    i0 = x0_ref[0, 0, :]
    i1 = x1_ref[0, 0, :]
    oh0 = (i0[:, None] == lax.broadcasted_iota(jnp.int32, (bm, V1), 1)
           ).astype(F32)
    oh1 = (i1[:, None] == lax.broadcasted_iota(jnp.int32, (bm, V2), 1)
           ).astype(F32)
    o_ref[...] = (jnp.dot(oh0, t1_ref[...], preferred_element_type=F32,
                          precision=jax.lax.Precision.HIGHEST)
                  + jnp.dot(oh1, t2_ref[...], preferred_element_type=F32,
                            precision=jax.lax.Precision.HIGHEST))

  return pl.pallas_call(
      body,
      grid=(NB,),
      in_specs=[
          pl.BlockSpec((1, 1, bm), lambda i: (i, 0, 0)),
          pl.BlockSpec((1, 1, bm), lambda i: (i, 0, 0)),
          pl.BlockSpec((V1, D), lambda i: (0, 0)),
          pl.BlockSpec((V2, D), lambda i: (0, 0)),
      ],
      out_specs=pl.BlockSpec((bm, D), lambda i: (i, 0)),
      out_shape=jax.ShapeDtypeStruct((NB * bm, D), F32),
  )(x0, x1, t1, t2)


def _gine_mm1(s01, s2a, s2b, h, c1, c2, e1p, e2p, eself, w1, b1):
  M, D = h.shape
  H = w1.shape[1]
  bm = _row_block(M)

  def body(s01_ref, s2a_ref, s2b_ref, h_ref, c1_ref, c2_ref,
           e1_ref, e2_ref, es_ref, w_ref, b_ref, o_ref):
    s2 = s2a_ref[...] + s2b_ref[...]
    scat = jnp.concatenate([s01_ref[...], s2[:, :D - 256]], axis=1)
    aggr = (scat + h_ref[...] + es_ref[...]
            + jnp.dot(c1_ref[...], e1_ref[...], preferred_element_type=F32,
                      precision=jax.lax.Precision.HIGHEST)
            + jnp.dot(c2_ref[...], e2_ref[...], preferred_element_type=F32,
                      precision=jax.lax.Precision.HIGHEST))
    y = jnp.dot(_rb(aggr), _rb(w_ref[...]),
                preferred_element_type=F32) + b_ref[...]
    o_ref[...] = jnp.maximum(y, 0.0)

  return pl.pallas_call(
      body,
      grid=(M // bm,),
      in_specs=[
          pl.BlockSpec((bm, 256), lambda i: (i, 0)),
          pl.BlockSpec((bm, 128), lambda i: (i, 0)),
          pl.BlockSpec((bm, 128), lambda i: (i, 0)),
          pl.BlockSpec((bm, D), lambda i: (i, 0)),
          pl.BlockSpec((bm, 16), lambda i: (i, 0)),
          pl.BlockSpec((bm, 16), lambda i: (i, 0)),
          pl.BlockSpec((16, D), lambda i: (0, 0)),
          pl.BlockSpec((16, D), lambda i: (0, 0)),
          pl.BlockSpec((1, D), lambda i: (0, 0)),
          pl.BlockSpec((D, H), lambda i: (0, 0)),
          pl.BlockSpec((1, H), lambda i: (0, 0)),
      ],
      out_specs=pl.BlockSpec((bm, H), lambda i: (i, 0)),
      out_shape=jax.ShapeDtypeStruct((M, H), F32),
  )(s01, s2a, s2b, h, c1, c2, e1p, e2p, eself.reshape(1, D), w1,
    b1.reshape(1, H))


def _gine_mm2(x, w2, b2):
  M, K = x.shape
  D = w2.shape[1]
  bm = _row_block(M)

  def body(x_ref, w_ref, b_ref, y_ref, s_ref):
    y = jnp.dot(_rb(x_ref[...]), _rb(w_ref[...]),
                preferred_element_type=F32) + b_ref[...]
    y_ref[...] = y

    @pl.when(pl.program_id(0) == 0)
    def _():
      s_ref[...] = jnp.zeros_like(s_ref)

    s_ref[...] += jnp.sum(y, axis=0, keepdims=True)

  return pl.pallas_call(
      body,
      grid=(M // bm,),
      in_specs=[
          pl.BlockSpec((bm, K), lambda i: (i, 0)),
          pl.BlockSpec((K, D), lambda i: (0, 0)),
          pl.BlockSpec((1, D), lambda i: (0, 0)),
      ],
      out_specs=[
          pl.BlockSpec((bm, D), lambda i: (i, 0)),
          pl.BlockSpec((1, D), lambda i: (0, 0)),
      ],
      out_shape=[
          jax.ShapeDtypeStruct((M, D), F32),
          jax.ShapeDtypeStruct((1, D), F32),
      ],
  )(x, w2, b2.reshape(1, D))


def _bn_var(x, ysum):
  M, D = x.shape
  bm = _row_block(M)
  inv_n = 1.0 / M

  def body(x_ref, s_ref, v_ref):
    mean = s_ref[...] * inv_n
    d = x_ref[...] - mean

    @pl.when(pl.program_id(0) == 0)
    def _():
      v_ref[...] = jnp.zeros_like(v_ref)

    v_ref[...] += jnp.sum(d * d, axis=0, keepdims=True)

  return pl.pallas_call(
      body,
      grid=(M // bm,),
      in_specs=[
          pl.BlockSpec((bm, D), lambda i: (i, 0)),
          pl.BlockSpec((1, D), lambda i: (0, 0)),
      ],
      out_specs=pl.BlockSpec((1, D), lambda i: (0, 0)),
      out_shape=jax.ShapeDtypeStruct((1, D), F32),
  )(x, ysum)


def _bn_apply(x, ysum, vsum, g, b, relu):
  M, D = x.shape
  bm = _row_block(M)
  inv_n = 1.0 / M

  def body(x_ref, s_ref, v_ref, g_ref, b_ref, o_ref):
    mean = s_ref[...] * inv_n
    var = v_ref[...] * inv_n
    scale = g_ref[...] / jnp.sqrt(var + EPS)
    y = (x_ref[...] - mean) * scale + b_ref[...]
    if relu:
      y = jnp.maximum(y, 0.0)
    o_ref[...] = y

  return pl.pallas_call(
      body,
      grid=(M // bm,),
      in_specs=[
          pl.BlockSpec((bm, D), lambda i: (i, 0)),
          pl.BlockSpec((1, D), lambda i: (0, 0)),
          pl.BlockSpec((1, D), lambda i: (0, 0)),
          pl.BlockSpec((1, D), lambda i: (0, 0)),
          pl.BlockSpec((1, D), lambda i: (0, 0)),
      ],
      out_specs=pl.BlockSpec((bm, D), lambda i: (i, 0)),
      out_shape=jax.ShapeDtypeStruct((M, D), F32),
  )(x, ysum, vsum, g.reshape(1, D), b.reshape(1, D))


def _inject(h, cinj, batch3, G):
  M, D = h.shape
  bm = _row_block(M)

  def body(h_ref, b_ref, c_ref, o_ref):
    bidx = b_ref[0, 0, :]
    oh = (bidx[:, None] == lax.broadcasted_iota(jnp.int32, (bm, G), 1)
          ).astype(F32)
    o_ref[...] = h_ref[...] + jnp.dot(oh, c_ref[...],
                                      preferred_element_type=F32,
                                      precision=jax.lax.Precision.HIGHEST)

  return pl.pallas_call(
      body,
      grid=(M // bm,),
      in_specs=[
          pl.BlockSpec((bm, D), lambda i: (i, 0)),
          pl.BlockSpec((1, 1, bm), lambda i: (i, 0, 0)),
          pl.BlockSpec((G, D), lambda i: (0, 0)),
      ],
      out_specs=pl.BlockSpec((bm, D), lambda i: (i, 0)),
      out_shape=jax.ShapeDtypeStruct((M, D), F32),
  )(h, batch3, cinj)


def _gat_ctx_dot(ctx, wdst, adst):
  G, D = ctx.shape

  def body(c_ref, w_ref, a_ref, o_ref):
    hd = jnp.dot(_rb(c_ref[...]), _rb(w_ref[...]), preferred_element_type=F32)
    o_ref[0, :] = jnp.sum(hd * a_ref[...], axis=1)

  return pl.pallas_call(
      body,
      grid=(1,),
      in_specs=[
          pl.BlockSpec((G, D), lambda i: (0, 0)),
          pl.BlockSpec((D, D), lambda i: (0, 0)),
          pl.BlockSpec((1, D), lambda i: (0, 0)),
      ],
      out_specs=pl.BlockSpec((1, G), lambda i: (0, 0)),
      out_shape=jax.ShapeDtypeStruct((1, G), F32),
  )(ctx, wdst, adst.reshape(1, D))


def _gat_logits(hs_ref, asrc_ref, s2_ref, bidx, bm, G):
  r = jnp.sum(hs_ref[...] * asrc_ref[...], axis=1)
  ohb = bidx[:, None] == lax.broadcasted_iota(jnp.int32, (bm, G), 1)
  s2g = jnp.sum(jnp.where(ohb, s2_ref[...], 0.0), axis=1)
  l = r + s2g
  return jnp.where(l >= 0, l, 0.2 * l), ohb


def _gat_pass1(hs, asrc, s2, batch3, G):
  M, D = hs.shape
  bm = _row_block(M)

  def body(hs_ref, a_ref, s2_ref, b_ref, m_ref):
    l, ohb = _gat_logits(hs_ref, a_ref, s2_ref, b_ref[0, 0, :], bm, G)

    @pl.when(pl.program_id(0) == 0)
    def _():
      m_ref[...] = jnp.full_like(m_ref, -jnp.inf)

    blk = jnp.max(jnp.where(ohb, l[:, None], -jnp.inf), axis=0)
    m_ref[0, :] = jnp.maximum(m_ref[0, :], blk)

  return pl.pallas_call(
      body,
      grid=(M // bm,),
      in_specs=[
          pl.BlockSpec((bm, D), lambda i: (i, 0)),
          pl.BlockSpec((1, D), lambda i: (0, 0)),
          pl.BlockSpec((1, G), lambda i: (0, 0)),
          pl.BlockSpec((1, 1, bm), lambda i: (i, 0, 0)),
      ],
      out_specs=pl.BlockSpec((1, G), lambda i: (0, 0)),
      out_shape=jax.ShapeDtypeStruct((1, G), F32),
  )(hs, asrc.reshape(1, D), s2, batch3)


def _gat_pass2(hs, asrc, s2, m, batch3, G):
  M, D = hs.shape
  bm = _row_block(M)

  def body(hs_ref, a_ref, s2_ref, m_ref, b_ref, num_ref, den_ref):
    bidx = b_ref[0, 0, :]
    l, ohb = _gat_logits(hs_ref, a_ref, s2_ref, bidx, bm, G)
    m = m_ref[...]
    mfin = jnp.where(jnp.isfinite(m), m, 0.0)
    mg = jnp.sum(jnp.where(ohb, mfin, 0.0), axis=1)
    ex = jnp.exp(l - mg)
    ohg = (bidx[None, :] == lax.broadcasted_iota(jnp.int32, (G, bm), 0)
           ).astype(F32)

    @pl.when(pl.program_id(0) == 0)
    def _():
      num_ref[...] = jnp.zeros_like(num_ref)
      den_ref[...] = jnp.zeros_like(den_ref)

    num_ref[...] += jnp.dot(ohg, ex[:, None] * hs_ref[...],
                            preferred_element_type=F32,
                            precision=jax.lax.Precision.HIGHEST)
    den_ref[0, :] += jnp.sum(jnp.where(ohb, ex[:, None], 0.0), axis=0)

  return pl.pallas_call(
      body,
      grid=(M // bm,),
      in_specs=[
          pl.BlockSpec((bm, D), lambda i: (i, 0)),
          pl.BlockSpec((1, D), lambda i: (0, 0)),
          pl.BlockSpec((1, G), lambda i: (0, 0)),
          pl.BlockSpec((1, G), lambda i: (0, 0)),
          pl.BlockSpec((1, 1, bm), lambda i: (i, 0, 0)),
      ],
      out_specs=[
          pl.BlockSpec((G, D), lambda i: (0, 0)),
          pl.BlockSpec((1, G), lambda i: (0, 0)),
      ],
      out_shape=[
          jax.ShapeDtypeStruct((G, D), F32),
          jax.ShapeDtypeStruct((1, G), F32),
      ],
  )(hs, asrc.reshape(1, D), s2, m, batch3)


def _gat_combine(numA, denA, numB, denB, gbias):
  G, D = numA.shape

  def body(na_ref, da_ref, nb_ref, db_ref, b_ref, o_ref):
    ca = na_ref[...] / (da_ref[...].reshape(G, 1) + 1e-16)
    cb = nb_ref[...] / (db_ref[...].reshape(G, 1) + 1e-16)
    o_ref[...] = jnp.maximum(ca + cb + 2.0 * b_ref[...], 0.0)

  return pl.pallas_call(
      body,
      grid=(1,),
      in_specs=[
          pl.BlockSpec((G, D), lambda i: (0, 0)),
          pl.BlockSpec((1, G), lambda i: (0, 0)),
          pl.BlockSpec((G, D), lambda i: (0, 0)),
          pl.BlockSpec((1, G), lambda i: (0, 0)),
          pl.BlockSpec((1, D), lambda i: (0, 0)),
      ],
      out_specs=pl.BlockSpec((G, D), lambda i: (0, 0)),
      out_shape=jax.ShapeDtypeStruct((G, D), F32),
  )(numA, denA, numB, denB, gbias.reshape(1, D))


def _pool(h, batch3, G):
  M, D = h.shape
  bm = _row_block(M)

  def body(h_ref, b_ref, s_ref, c_ref):
    bidx = b_ref[0, 0, :]
    ohg = (bidx[None, :] == lax.broadcasted_iota(jnp.int32, (G, bm), 0)
           ).astype(F32)

    @pl.when(pl.program_id(0) == 0)
    def _():
      s_ref[...] = jnp.zeros_like(s_ref)
      c_ref[...] = jnp.zeros_like(c_ref)

    s_ref[...] += jnp.dot(ohg, h_ref[...], preferred_element_type=F32,
                          precision=jax.lax.Precision.HIGHEST)
    c_ref[0, :] += jnp.sum(ohg, axis=1)

  return pl.pallas_call(
      body,
      grid=(M // bm,),
      in_specs=[
          pl.BlockSpec((bm, D), lambda i: (i, 0)),
          pl.BlockSpec((1, 1, bm), lambda i: (i, 0, 0)),
      ],
      out_specs=[
          pl.BlockSpec((G, D), lambda i: (0, 0)),
          pl.BlockSpec((1, G), lambda i: (0, 0)),
      ],
      out_shape=[
          jax.ShapeDtypeStruct((G, D), F32),
          jax.ShapeDtypeStruct((1, G), F32),
      ],
  )(h, batch3)


def _mm_meanpool(sums, cnt, w, b):
  G, D = sums.shape
  Nc = w.shape[1]

  def body(s_ref, c_ref, w_ref, b_ref, o_ref):
    rs = 1.0 / jnp.maximum(c_ref[...], 1.0)
    x = s_ref[...] * rs.reshape(G, 1)
    y = jnp.dot(_rb(x), _rb(w_ref[...]), preferred_element_type=F32) + b_ref[...]
    o_ref[...] = jnp.maximum(y, 0.0)

  return pl.pallas_call(
      body,
      grid=(1,),
      in_specs=[
          pl.BlockSpec((G, D), lambda i: (0, 0)),
          pl.BlockSpec((1, G), lambda i: (0, 0)),
          pl.BlockSpec((D, Nc), lambda i: (0, 0)),
          pl.BlockSpec((1, Nc), lambda i: (0, 0)),
      ],
      out_specs=pl.BlockSpec((G, Nc), lambda i: (0, 0)),
      out_shape=jax.ShapeDtypeStruct((G, Nc), F32),
  )(sums, cnt, w, b.reshape(1, Nc))


def _gine(h, src, dst, c1, c2, e1, e2, w1, b1, w2, b2, g, b, relu):
  s01, s2a, s2b = _sc_scatter(h, src, dst)
  e1p = jnp.pad(e1, ((0, 16 - e1.shape[0]), (0, 0)))
  e2p = jnp.pad(e2, ((0, 16 - e2.shape[0]), (0, 0)))
  eself = e1[4] + e2[0]
  y = _gine_mm1(s01, s2a, s2b, h, c1, c2, e1p, e2p, eself, w1, b1)
  y, ysum = _gine_mm2(y, w2, b2)
  vsum = _bn_var(y, ysum)
  return _bn_apply(y, ysum, vsum, g, b, relu)


def _gat(h, ctx, batch3, wsrc, wdst, asrc, adst, G):
  hs = _mm(h, wsrc, jnp.zeros((wsrc.shape[1],), F32))
  s2 = _gat_ctx_dot(ctx, wdst, adst)
  m = _gat_pass1(hs, asrc, s2, batch3, G)
  return _gat_pass2(hs, asrc, s2, m, batch3, G)


def kernel(xA, edge_indexA, edge_attrA, batchA, xB, edge_indexB, edge_attrB,
           batchB, context, params):
  p = params
  N = xA.shape[0]
  G = context.shape[0]
  bm = _row_block(N)
  NB = N // bm

  bA3 = batchA.reshape(NB, 1, bm)
  bB3 = batchB.reshape(NB, 1, bm)
  x0A = xA[:, 0].reshape(NB, 1, bm)
  x1A = xA[:, 1].reshape(NB, 1, bm)
  x0B = xB[:, 0].reshape(NB, 1, bm)
  x1B = xB[:, 1].reshape(NB, 1, bm)

  hA = _embed(x0A, x1A, p['x_emb1'], p['x_emb2'])
  hB = _embed(x0B, x1B, p['x_emb1'], p['x_emb2'])

  ctx = _mm(context, p['Wc1'], p['bc1'], act="relu")
  ctx = _mm(ctx, p['Wc2'], p['bc2'])

  srcA, dstA = edge_indexA[0], edge_indexA[1]
  srcB, dstB = edge_indexB[0], edge_indexB[1]
  c1A, c2A = _sc_hist(edge_attrA[:, 0], edge_attrA[:, 1], dstA, N)
  c1B, c2B = _sc_hist(edge_attrB[:, 0], edge_attrB[:, 1], dstB, N)

  nb = p['b_W1'].shape[0]
  for i in range(nb):
    relu = i != nb - 1
    hA = _gine(hA, srcA, dstA, c1A, c2A, p['b_e1'][i], p['b_e2'][i],
               p['b_W1'][i], p['b_b1'][i], p['b_W2'][i], p['b_b2'][i],
               p['b_bn_g'][i], p['b_bn_b'][i], relu)
    hB = _gine(hB, srcB, dstB, c1B, c2B, p['b_e1'][i], p['b_e2'][i],
               p['b_W1'][i], p['b_b1'][i], p['b_W2'][i], p['b_b2'][i],
               p['b_bn_g'][i], p['b_bn_b'][i], relu)

  nc = p['c_W1'].shape[0]
  for i in range(nc):
    relu = i != nc - 1
    cinj = _mm(ctx, p['c_injW'][i], p['c_injb'][i])
    hA = _inject(hA, cinj, bA3, G)
    hB = _inject(hB, cinj, bB3, G)
    hA = _gine(hA, srcA, dstA, c1A, c2A, p['c_e1'][i], p['c_e2'][i],
               p['c_W1'][i], p['c_b1'][i], p['c_W2'][i], p['c_b2'][i],
               p['c_bn_g'][i], p['c_bn_b'][i], relu)
    hB = _gine(hB, srcB, dstB, c1B, c2B, p['c_e1'][i], p['c_e2'][i],
               p['c_W1'][i], p['c_b1'][i], p['c_W2'][i], p['c_b2'][i],
               p['c_bn_g'][i], p['c_bn_b'][i], relu)
    numA, denA = _gat(hA, ctx, bA3, p['g_Wsrc'][i], p['g_Wdst'][i],
                      p['g_asrc'][i], p['g_adst'][i], G)
    numB, denB = _gat(hB, ctx, bB3, p['g_Wsrc'][i], p['g_Wdst'][i],
                      p['g_asrc'][i], p['g_adst'][i], G)
    ctx = _gat_combine(numA, denA, numB, denB, p['g_b'][i])

  sumsA, cntA = _pool(hA, bA3, G)
  gA = _mm_meanpool(sumsA, cntA, p['Wo1'], p['bo1'])
  gA = _mm(gA, p['Wo2'], p['bo2'])
  sumsB, cntB = _pool(hB, bB3, G)
  gB = _mm_meanpool(sumsB, cntB, p['Wo1'], p['bo1'])
  gB = _mm(gB, p['Wo2'], p['bo2'])

  h = jnp.concatenate([gA, gB, ctx], axis=1)
  h = _mm(h, p['Wm1'], p['bm1'], act="leaky")
  h = _mm(h, p['Wm2'], p['bm2'], act="leaky")
  h = _mm(h, p['Wm3'], p['bm3'], act="leaky")
  return _mm(h, p['Wm4'], p['bm4'])
